# Initial kernel scaffold; baseline (speedup 1.0000x reference)
import jax
import jax.numpy as jnp
from jax.experimental import pallas as pl


def kernel(node_features, edge_index, edge_types, W_rel1, W_self1, b1, g1, be1, W_rel2, W_self2, b2, g2, be2):
    raise NotImplementedError("write your pallas kernel here")


# speedup vs baseline: 1.8208x; 1.8208x over previous
import functools

import jax
import jax.numpy as jnp
from jax import lax
from jax.experimental import pallas as pl
from jax.experimental.pallas import tpu as pltpu
from jax.experimental.pallas import tpu_sc as plsc

N = 10000
E = 320000
D = 128
R = 16
EPS = 1e-5

NC = 2
NS = 16
NW = NC * NS
EPT = E // NW
CH = 80
NCHK = EPT // CH
SLAB = 25
NSLAB = NCHK // SLAB
LANE = 16

BN = 1000
NB = N // BN


def _mm_body(x_ref, w_ref, o_ref):
    o_ref[0] = jnp.dot(x_ref[...], w_ref[0], preferred_element_type=jnp.float32)


def _xw(x, w_all):
    return pl.pallas_call(
        _mm_body,
        grid=(NB, R + 1),
        in_specs=[
            pl.BlockSpec((BN, D), lambda nb, r: (nb, 0)),
            pl.BlockSpec((1, D, D), lambda nb, r: (r, 0, 0)),
        ],
        out_specs=pl.BlockSpec((1, BN, D), lambda nb, r: (r, nb, 0)),
        out_shape=jax.ShapeDtypeStruct((R + 1, N, D), jnp.float32),
    )(x, w_all)


def _make_sc_agg():
    mesh = plsc.VectorSubcoreMesh(core_axis_name="c", subcore_axis_name="s")

    out_type = jax.ShapeDtypeStruct((NC, N, D), jnp.float32)

    scratch = [
        pltpu.VMEM((CH,), jnp.int32),
        pltpu.VMEM((CH,), jnp.int32),
        pltpu.VMEM((CH, D), jnp.float32),
        pltpu.SemaphoreType.DMA,
        pltpu.VMEM_SHARED((N, D), jnp.float32),
    ]

    def body(xw_hbm, gidx_hbm, dst_hbm, z_hbm,
             agg_out, gidxb, dstb, rowb, sem, agg_sh):
        c = lax.axis_index("c")
        s = lax.axis_index("s")
        w = c * NS + s

        @pl.when(s == 0)
        def _():
            pltpu.sync_copy(z_hbm, agg_sh)

        plsc.subcore_barrier()

        def _step(ci, _):
            pltpu.sync_copy(gidx_hbm.at[w, ci], gidxb)
            pltpu.sync_copy(dst_hbm.at[w, ci], dstb)
            pltpu.async_copy(xw_hbm.at[gidxb], rowb, sem).wait()
            pltpu.sync_copy(rowb, agg_sh.at[dstb], add=True)
            return 0
        lax.fori_loop(0, NCHK, _step, 0)

        plsc.subcore_barrier()

        @pl.when(s == 0)
        def _():
            pltpu.sync_copy(agg_sh, agg_out.at[c])

    return pl.kernel(body, out_type=out_type, mesh=mesh,
                     scratch_types=scratch)


_make_sc_agg = functools.lru_cache(maxsize=None)(_make_sc_agg)


def _sc_agg(*args):
    return _make_sc_agg()(*args)


def _comb_body(hs_ref, a_ref, rd_ref, b_ref, g_ref, be_ref, o_ref):
    a = a_ref[0] + a_ref[1]
    h = hs_ref[...] + a * rd_ref[...] + b_ref[0]
    h = jnp.maximum(h, 0.0)
    mu = jnp.mean(h, axis=1, keepdims=True)
    var = jnp.mean((h - mu) ** 2, axis=1, keepdims=True)
    o_ref[...] = (h - mu) / jnp.sqrt(var + EPS) * g_ref[0] + be_ref[0]


def _combine(hself, agg2, rdegb, b, g, be):
    return pl.pallas_call(
        _comb_body,
        grid=(NB,),
        in_specs=[
            pl.BlockSpec((BN, D), lambda nb: (nb, 0)),
            pl.BlockSpec((NC, BN, D), lambda nb: (0, nb, 0)),
            pl.BlockSpec((BN, D), lambda nb: (nb, 0)),
            pl.BlockSpec((1, D), lambda nb: (0, 0)),
            pl.BlockSpec((1, D), lambda nb: (0, 0)),
            pl.BlockSpec((1, D), lambda nb: (0, 0)),
        ],
        out_specs=pl.BlockSpec((BN, D), lambda nb: (nb, 0)),
        out_shape=jax.ShapeDtypeStruct((N, D), jnp.float32),
    )(hself, agg2, rdegb, b.reshape(1, D), g.reshape(1, D), be.reshape(1, D))


def kernel(node_features, edge_index, edge_types,
           W_rel1, W_self1, b1, g1, be1,
           W_rel2, W_self2, b2, g2, be2):
    gidx4 = (edge_types * N + edge_index[0]).reshape(NW, NCHK, CH)
    dst4 = edge_index[1].reshape(NW, NCHK, CH)
    zros = jnp.zeros((N, D), jnp.float32)

    deg = jax.ops.segment_sum(jnp.ones((E,), jnp.float32), edge_index[1],
                              num_segments=N)
    rdegb = jnp.broadcast_to((1.0 / jnp.maximum(deg, 1.0))[:, None], (N, D))

    w_all1 = jnp.concatenate([W_rel1, W_self1[None]], axis=0)
    xw1 = _xw(node_features, w_all1)
    agg1 = _sc_agg(xw1.reshape((R + 1) * N, D), gidx4, dst4, zros)
    h1 = _combine(xw1[R], agg1, rdegb, b1, g1, be1)

    w_all2 = jnp.concatenate([W_rel2, W_self2[None]], axis=0)
    xw2 = _xw(h1, w_all2)
    agg2 = _sc_agg(xw2.reshape((R + 1) * N, D), gidx4, dst4, zros)
    h2 = _combine(xw2[R], agg2, rdegb, b2, g2, be2)
    return h2
